# baseline (speedup 1.0000x reference)
import dataclasses
import functools

import jax
import jax.numpy as jnp
from jax import lax
from jax.experimental import pallas as pl
from jax.experimental.pallas import tpu as pltpu
from jax.experimental.pallas import tpu_sc as plsc

_NC = 2
_NS = 16
_LANES = 16
_CH = 128
_SB = 16


def _sc_aggregate_fn(n_nodes, d, ept, npad):
    nslice = d // _LANES
    slab = npad // _NS
    zk = slab // _CH
    nchunks = ept // _CH
    nsc = nchunks // _SB
    mesh = plsc.VectorSubcoreMesh(core_axis_name="c", subcore_axis_name="s")
    cp = pltpu.CompilerParams()
    if "needs_layout_passes" in pltpu.CompilerParams.__dataclass_fields__:
        cp = dataclasses.replace(cp, needs_layout_passes=False)

    @functools.partial(
        pl.kernel,
        compiler_params=cp,
        out_type=jax.ShapeDtypeStruct((_NC, npad, d), jnp.float32),
        mesh=mesh,
        scratch_types=[
            pltpu.VMEM((_SB, _CH), jnp.int32),
            pltpu.VMEM((_SB, _CH), jnp.int32),
            pltpu.VMEM((_SB, _CH), jnp.int32),
            pltpu.VMEM((_SB, _CH), jnp.int32),
            pltpu.VMEM((_SB * _CH,), jnp.float32),
            pltpu.VMEM((_SB * _CH,), jnp.float32),
            pltpu.VMEM((_CH, d), jnp.float32),
            pltpu.VMEM((_CH, d), jnp.float32),
            pltpu.VMEM((d,), jnp.float32),
            pltpu.VMEM((d,), jnp.float32),
            pltpu.VMEM_SHARED((npad, d), jnp.float32),
            pltpu.SemaphoreType.DMA,
            pltpu.SemaphoreType.DMA,
        ],
    )
    def body(h_hbm, src_hbm, dst_hbm, attr_hbm, w_hbm, b_hbm, out_hbm,
             src0, src1, dst0, dst1, at0, at1, rows0, rows1, w_v, b_v,
             acc, sem, sem2):
        cid = lax.axis_index("c")
        sid = lax.axis_index("s")
        gid = cid * _NS + sid

        pltpu.sync_copy(w_hbm, w_v)
        pltpu.sync_copy(b_hbm, b_v)

        ws = [w_v[pl.ds(t * _LANES, _LANES)] for t in range(nslice)]
        bs = [b_v[pl.ds(t * _LANES, _LANES)] for t in range(nslice)]

        zvec = jnp.zeros((_LANES,), jnp.float32)

        @pl.loop(0, _CH)
        def _(r):
            for t in range(nslice):
                rows0[r, pl.ds(t * _LANES, _LANES)] = zvec

        for k in range(zk):
            pltpu.sync_copy(rows0, acc.at[pl.ds(sid * slab + k * _CH, _CH)])
        plsc.subcore_barrier()

        sbufs = ((src0, dst0, at0), (src1, dst1, at1))

        def stage_refs(q, bufs):
            s_v, d_v, a_v = bufs
            return (
                (src_hbm.at[pl.ds(gid * nchunks + q * _SB, _SB)], s_v),
                (dst_hbm.at[pl.ds(gid * nchunks + q * _SB, _SB)], d_v),
                (attr_hbm.at[pl.ds(gid * ept + q * _SB * _CH, _SB * _CH)],
                 a_v),
            )

        for s, t in stage_refs(0, sbufs[0]):
            pltpu.async_copy(s, t, sem2)
        for q in range(nsc):
            s_v, d_v, a_v = sbufs[q % 2]
            for s, t in stage_refs(q, sbufs[q % 2]):
                pltpu.make_async_copy(s, t, sem2).wait()
            if q + 1 < nsc:
                for s, t in stage_refs(q + 1, sbufs[(q + 1) % 2]):
                    pltpu.async_copy(s, t, sem2)

            pltpu.async_copy(h_hbm.at[s_v.at[0]], rows0, sem)

            @pl.loop(0, _SB, step=2)
            def _(t):
                for bno, buf, obuf in ((0, rows0, rows1), (1, rows1, rows0)):
                    cur = t + bno
                    pltpu.make_async_copy(
                        h_hbm.at[s_v.at[cur]], buf, sem).wait()

                    @pl.when(cur + 1 < _SB)
                    def _():
                        pltpu.async_copy(h_hbm.at[s_v.at[cur + 1]], obuf, sem)

                    cbase = cur * _CH

                    @pl.loop(0, _CH, step=4)
                    def _(j):
                        for u in range(4):
                            jj = j + u
                            a_vec = plsc.load_gather(
                                a_v,
                                [jnp.full((_LANES,), cbase + jj, jnp.int32)])
                            for t2 in range(nslice):
                                sl = pl.ds(t2 * _LANES, _LANES)
                                buf[jj, sl] = jnp.maximum(
                                    buf[jj, sl] + (a_vec * ws[t2] + bs[t2]),
                                    0.0)

                    pltpu.sync_copy(buf, acc.at[d_v.at[cur]], add=True)

        plsc.subcore_barrier()
        for k in range(zk):
            r0 = sid * slab + k * _CH
            pltpu.sync_copy(acc.at[pl.ds(r0, _CH)],
                            out_hbm.at[cid, pl.ds(r0, _CH)])

    return body


def _tc_update(h, parts, w1t, b1, w2t, b2, scale, block_n):
    n, d = h.shape

    def body(scale_ref, h_ref, p_ref, w1_ref, b1_ref, w2_ref, b2_ref, o_ref):
        s = scale_ref[0]
        z = s * h_ref[...] + p_ref[0] + p_ref[1]
        hmid = jnp.maximum(
            jnp.dot(z, w1_ref[...], preferred_element_type=jnp.float32)
            + b1_ref[...], 0.0)
        o_ref[...] = jnp.maximum(
            jnp.dot(hmid, w2_ref[...], preferred_element_type=jnp.float32)
            + b2_ref[...], 0.0)

    grid = (n // block_n,)
    return pl.pallas_call(
        body,
        grid=grid,
        in_specs=[
            pl.BlockSpec(memory_space=pltpu.SMEM),
            pl.BlockSpec((block_n, d), lambda i: (i, 0)),
            pl.BlockSpec((_NC, block_n, d), lambda i: (0, i, 0)),
            pl.BlockSpec((d, d), lambda i: (0, 0)),
            pl.BlockSpec((1, d), lambda i: (0, 0)),
            pl.BlockSpec((d, d), lambda i: (0, 0)),
            pl.BlockSpec((1, d), lambda i: (0, 0)),
        ],
        out_specs=pl.BlockSpec((block_n, d), lambda i: (i, 0)),
        out_shape=jax.ShapeDtypeStruct((n, d), jnp.float32),
    )(scale, h, parts, w1t, b1, w2t, b2)


def kernel(x, edge_index, edge_attr, We, be, W1, b1, W2, b2, eps):
    n, d = x.shape
    e = edge_index.shape[1]
    nlayers = We.shape[0]

    nw = _NC * _NS
    nchunks = -(-(-(-e // (nw * _CH))) // _SB) * _SB
    ept = nchunks * _CH
    ep = ept * nw
    npad = -(-(n + 1) // (_NS * _CH)) * (_NS * _CH)

    pad = ep - e
    src = jnp.concatenate(
        [edge_index[0], jnp.zeros((pad,), jnp.int32)]).reshape(-1, _CH)
    dst = jnp.concatenate(
        [edge_index[1], jnp.full((pad,), n, jnp.int32)]).reshape(-1, _CH)
    attr = jnp.concatenate([edge_attr, jnp.zeros((pad,), jnp.float32)])

    sc_fn = _sc_aggregate_fn(n, d, ept, npad)

    h = x
    outs = []
    for l in range(nlayers):
        parts = sc_fn(h, src, dst, attr, We[l][:, 0], be[l])
        h = _tc_update(h, parts[:, :n, :], W1[l].T, b1[l][None, :],
                       W2[l].T, b2[l][None, :],
                       (1.0 + eps[l])[None], block_n=1000)
        outs.append(h)
    return jnp.stack(outs)


# speedup vs baseline: 1.1633x; 1.1633x over previous
import dataclasses
import functools

import jax
import jax.numpy as jnp
from jax import lax
from jax.experimental import pallas as pl
from jax.experimental.pallas import tpu as pltpu
from jax.experimental.pallas import tpu_sc as plsc

_NC = 2
_NS = 16
_LANES = 16
_CH = 64
_SB = 16
_NB = 4


def _sc_aggregate_fn(n_nodes, d, ept, npad):
    nslice = d // _LANES
    slab = npad // _NS
    zk = slab // _CH
    nchunks = ept // _CH
    nsc = nchunks // _SB
    mesh = plsc.VectorSubcoreMesh(core_axis_name="c", subcore_axis_name="s")
    cp = pltpu.CompilerParams()
    if "needs_layout_passes" in pltpu.CompilerParams.__dataclass_fields__:
        cp = dataclasses.replace(cp, needs_layout_passes=False)

    @functools.partial(
        pl.kernel,
        compiler_params=cp,
        out_type=jax.ShapeDtypeStruct((_NC, npad, d), jnp.float32),
        mesh=mesh,
        scratch_types=[
            pltpu.VMEM((_SB, _CH), jnp.int32),
            pltpu.VMEM((_SB, _CH), jnp.int32),
            pltpu.VMEM((_SB, _CH), jnp.int32),
            pltpu.VMEM((_SB, _CH), jnp.int32),
            pltpu.VMEM((_SB * _CH,), jnp.float32),
            pltpu.VMEM((_SB * _CH,), jnp.float32),
            pltpu.VMEM((_CH, d), jnp.float32),
            pltpu.VMEM((_CH, d), jnp.float32),
            pltpu.VMEM((_CH, d), jnp.float32),
            pltpu.VMEM((_CH, d), jnp.float32),
            pltpu.VMEM((d,), jnp.float32),
            pltpu.VMEM((d,), jnp.float32),
            pltpu.VMEM_SHARED((npad, d), jnp.float32),
            pltpu.SemaphoreType.DMA,
            pltpu.SemaphoreType.DMA,
            pltpu.SemaphoreType.DMA,
        ],
    )
    def body(h_hbm, src_hbm, dst_hbm, attr_hbm, w_hbm, b_hbm, out_hbm,
             src0, src1, dst0, dst1, at0, at1, rows0, rows1, rows2, rows3,
             w_v, b_v, acc, sem, sem2, sem3):
        cid = lax.axis_index("c")
        sid = lax.axis_index("s")
        gid = cid * _NS + sid
        rows = (rows0, rows1, rows2, rows3)

        pltpu.sync_copy(w_hbm, w_v)
        pltpu.sync_copy(b_hbm, b_v)

        ws = [w_v[pl.ds(t * _LANES, _LANES)] for t in range(nslice)]
        bs = [b_v[pl.ds(t * _LANES, _LANES)] for t in range(nslice)]

        zvec = jnp.zeros((_LANES,), jnp.float32)

        @pl.loop(0, _CH)
        def _(r):
            for t in range(nslice):
                rows0[r, pl.ds(t * _LANES, _LANES)] = zvec

        for k in range(zk):
            pltpu.sync_copy(rows0, acc.at[pl.ds(sid * slab + k * _CH, _CH)])
        plsc.subcore_barrier()

        sbufs = ((src0, dst0, at0), (src1, dst1, at1))

        def stage_refs(q, bufs):
            s_v, d_v, a_v = bufs
            return (
                (src_hbm.at[pl.ds(gid * nchunks + q * _SB, _SB)], s_v),
                (dst_hbm.at[pl.ds(gid * nchunks + q * _SB, _SB)], d_v),
                (attr_hbm.at[pl.ds(gid * ept + q * _SB * _CH, _SB * _CH)],
                 a_v),
            )

        def wait_scatter(d_v):
            pltpu.make_async_copy(rows0, acc.at[d_v.at[0]], sem3).wait()

        def process(s_v, d_v, a_v, cur, u, drain):
            buf = rows[u]
            pltpu.make_async_copy(h_hbm.at[s_v.at[cur]], buf, sem).wait()
            cbase = cur * _CH

            @pl.loop(0, _CH, step=4)
            def _(j):
                for v in range(4):
                    jj = j + v
                    a_vec = plsc.load_gather(
                        a_v, [jnp.full((_LANES,), cbase + jj, jnp.int32)])
                    for t2 in range(nslice):
                        sl = pl.ds(t2 * _LANES, _LANES)
                        buf[jj, sl] = jnp.maximum(
                            buf[jj, sl] + (a_vec * ws[t2] + bs[t2]), 0.0)

            pltpu.async_copy(buf, acc.at[d_v.at[cur]], sem3, add=True)
            if drain:
                wait_scatter(d_v)
            nbuf = rows[(u + 2) % _NB]
            if isinstance(cur, int):
                if cur + 2 < _SB:
                    pltpu.async_copy(h_hbm.at[s_v.at[cur + 2]], nbuf, sem)
            else:
                @pl.when(cur + 2 < _SB)
                def _():
                    pltpu.async_copy(h_hbm.at[s_v.at[cur + 2]], nbuf, sem)

        for s, t in stage_refs(0, sbufs[0]):
            pltpu.async_copy(s, t, sem2)
        for q in range(nsc):
            s_v, d_v, a_v = sbufs[q % 2]
            for s, t in stage_refs(q, sbufs[q % 2]):
                pltpu.make_async_copy(s, t, sem2).wait()
            if q + 1 < nsc:
                for s, t in stage_refs(q + 1, sbufs[(q + 1) % 2]):
                    pltpu.async_copy(s, t, sem2)

            pltpu.async_copy(h_hbm.at[s_v.at[0]], rows[0], sem)
            pltpu.async_copy(h_hbm.at[s_v.at[1]], rows[1], sem)

            if q == 0:
                for u in range(_NB):
                    process(s_v, d_v, a_v, u, u, drain=u >= 2)
                t0 = _NB
            else:
                t0 = 0

            @pl.loop(t0, _SB, step=_NB)
            def _(t):
                for u in range(_NB):
                    process(s_v, d_v, a_v, t + u, u, drain=True)

        wait_scatter(sbufs[(nsc - 1) % 2][1])
        wait_scatter(sbufs[(nsc - 1) % 2][1])

        plsc.subcore_barrier()
        for k in range(zk):
            r0 = sid * slab + k * _CH
            pltpu.sync_copy(acc.at[pl.ds(r0, _CH)],
                            out_hbm.at[cid, pl.ds(r0, _CH)])

    return body


def _tc_update(h, parts, w1t, b1, w2t, b2, scale, block_n):
    n, d = h.shape

    def body(scale_ref, h_ref, p_ref, w1_ref, b1_ref, w2_ref, b2_ref, o_ref):
        s = scale_ref[0]
        z = s * h_ref[...] + p_ref[0] + p_ref[1]
        hmid = jnp.maximum(
            jnp.dot(z, w1_ref[...], preferred_element_type=jnp.float32)
            + b1_ref[...], 0.0)
        o_ref[...] = jnp.maximum(
            jnp.dot(hmid, w2_ref[...], preferred_element_type=jnp.float32)
            + b2_ref[...], 0.0)

    grid = (n // block_n,)
    return pl.pallas_call(
        body,
        grid=grid,
        in_specs=[
            pl.BlockSpec(memory_space=pltpu.SMEM),
            pl.BlockSpec((block_n, d), lambda i: (i, 0)),
            pl.BlockSpec((_NC, block_n, d), lambda i: (0, i, 0)),
            pl.BlockSpec((d, d), lambda i: (0, 0)),
            pl.BlockSpec((1, d), lambda i: (0, 0)),
            pl.BlockSpec((d, d), lambda i: (0, 0)),
            pl.BlockSpec((1, d), lambda i: (0, 0)),
        ],
        out_specs=pl.BlockSpec((block_n, d), lambda i: (i, 0)),
        out_shape=jax.ShapeDtypeStruct((n, d), jnp.float32),
    )(scale, h, parts, w1t, b1, w2t, b2)


def kernel(x, edge_index, edge_attr, We, be, W1, b1, W2, b2, eps):
    n, d = x.shape
    e = edge_index.shape[1]
    nlayers = We.shape[0]

    nw = _NC * _NS
    nchunks = -(-(-(-e // (nw * _CH))) // _SB) * _SB
    ept = nchunks * _CH
    ep = ept * nw
    npad = -(-(n + 1) // (_NS * _CH)) * (_NS * _CH)

    pad = ep - e
    src = jnp.concatenate(
        [edge_index[0], jnp.zeros((pad,), jnp.int32)]).reshape(-1, _CH)
    dst_pad = n + jnp.arange(pad, dtype=jnp.int32) % (npad - n)
    dst = jnp.concatenate([edge_index[1], dst_pad]).reshape(-1, _CH)
    attr = jnp.concatenate([edge_attr, jnp.zeros((pad,), jnp.float32)])

    sc_fn = _sc_aggregate_fn(n, d, ept, npad)

    h = x
    outs = []
    for l in range(nlayers):
        parts = sc_fn(h, src, dst, attr, We[l][:, 0], be[l])
        h = _tc_update(h, parts[:, :n, :], W1[l].T, b1[l][None, :],
                       W2[l].T, b2[l][None, :],
                       (1.0 + eps[l])[None], block_n=1000)
        outs.append(h)
    return jnp.stack(outs)
